# baseline (speedup 1.0000x reference)
import functools

import jax
import jax.numpy as jnp
from jax import lax
from jax.experimental import pallas as pl
from jax.experimental.pallas import tpu as pltpu
from jax.experimental.pallas import tpu_sc as plsc

_K = 1024
_D = 64
_N = 4608
_R = 2304
_NC = 2
_NS = 16
_BW = _N // (_NC * _NS)


def _argmin_block(x_ref, cb_ref, tok_ref, cbp_ref):
    x = x_ref[...]
    cb = cb_ref[...]

    @pl.when(pl.program_id(0) == 0)
    def _():
        cbp_ref[:, : _D] = cb

    c2 = jnp.sum(cb * cb, axis=1)
    v2 = jnp.sum(x * x, axis=1, keepdims=True)
    cross2 = jax.lax.dot_general(
        x, cb + cb, (((1,), (1,)), ((), ())),
        preferred_element_type=jnp.float32)
    dist = v2 - cross2 + c2[None, :]
    tok = jnp.argmin(dist, axis=1).astype(jnp.int32)
    tok_ref[...] = tok[:, None]


def _sc_gather_body(cb_hbm, idx_hbm, out_hbm, idx_v, rows_v, sem):
    wid = lax.axis_index("s") * _NC + lax.axis_index("c")
    base = wid * _BW
    pltpu.sync_copy(idx_hbm.at[pl.ds(base, _BW)], idx_v)
    pltpu.async_copy(cb_hbm.at[idx_v], rows_v, sem).wait()
    pltpu.sync_copy(rows_v, out_hbm.at[pl.ds(base, _BW)])


_sc_gather = functools.partial(
    pl.kernel,
    out_type=jax.ShapeDtypeStruct((_N, 128), jnp.float32),
    mesh=plsc.VectorSubcoreMesh(core_axis_name="c", subcore_axis_name="s"),
    scratch_types=[
        pltpu.VMEM((_BW,), jnp.int32),
        pltpu.VMEM((_BW, 128), jnp.float32),
        pltpu.SemaphoreType.DMA,
    ],
)(_sc_gather_body)


def kernel(inputs, codebook, training):
    del training
    b, h, w, d = inputs.shape
    x = inputs.reshape(_N, d)
    tok, cb128 = pl.pallas_call(
        _argmin_block,
        grid=(_N // _R,),
        in_specs=[
            pl.BlockSpec((_R, d), lambda i: (i, 0)),
            pl.BlockSpec((_K, d), lambda i: (0, 0)),
        ],
        out_specs=[
            pl.BlockSpec((_R, 1), lambda i: (i, 0)),
            pl.BlockSpec((_K, 128), lambda i: (0, 0)),
        ],
        out_shape=[
            jax.ShapeDtypeStruct((_N, 1), jnp.int32),
            jax.ShapeDtypeStruct((_K, 128), jnp.float32),
        ],
    )(x, codebook)
    out = _sc_gather(cb128, tok.reshape(_N))
    return out[:, :_D].reshape(b, h, w, d)


# speedup vs baseline: 2.0973x; 2.0973x over previous
import jax
import jax.numpy as jnp
from jax.experimental import pallas as pl
from jax.experimental.pallas import tpu as pltpu

_K = 1024
_D = 64
_R = 4608


def _vq_block(x_ref, cb_ref, out_ref):
    x = x_ref[...]
    cb = cb_ref[...]
    c2 = jnp.sum(cb * cb, axis=1)
    v2 = jnp.sum(x * x, axis=1, keepdims=True)
    cross2 = jax.lax.dot_general(
        x, cb + cb, (((1,), (1,)), ((), ())),
        preferred_element_type=jnp.float32)
    dist = v2 - cross2 + c2[None, :]
    tok = jnp.argmin(dist, axis=1).astype(jnp.int32)
    iota = jax.lax.broadcasted_iota(jnp.int32, (_R, _K), 1)
    onehot = (iota == tok[:, None]).astype(jnp.float32)
    emb = jax.lax.dot_general(
        onehot, cb, (((1,), (0,)), ((), ())),
        preferred_element_type=jnp.float32)
    out_ref[...] = x + (emb - x)


def kernel(inputs, codebook, training):
    del training
    b, h, w, d = inputs.shape
    n = b * h * w
    x = inputs.reshape(n, d)
    out = pl.pallas_call(
        _vq_block,
        grid=(n // _R,),
        in_specs=[
            pl.BlockSpec((_R, d), lambda i: (i, 0)),
            pl.BlockSpec((_K, d), lambda i: (0, 0)),
        ],
        out_specs=pl.BlockSpec((_R, d), lambda i: (i, 0)),
        out_shape=jax.ShapeDtypeStruct((n, d), jnp.float32),
    )(x, codebook)
    return out.reshape(b, h, w, d)
